# Initial kernel scaffold; baseline (speedup 1.0000x reference)
import jax
import jax.numpy as jnp
from jax.experimental import pallas as pl


def kernel(x, pos, batch, W1, b1, W2, b2, Wfc, bfc):
    raise NotImplementedError("write your pallas kernel here")


# speedup vs baseline: 4.4092x; 4.4092x over previous
import functools

import jax
import jax.numpy as jnp
from jax import lax
from jax.experimental import pallas as pl
from jax.experimental.pallas import tpu as pltpu
from jax.experimental.pallas import tpu_sc as plsc

N = 100000
F = 128
G = 1024
SEG_PER_W = 32
C = 128
RB = 800
NB = N // RB


def _score_body(x_ref, w1_ref, b1_ref, w2_ref, b2_ref, o_ref):
    h = jnp.dot(x_ref[...], w1_ref[...], preferred_element_type=jnp.float32)
    h = h + b1_ref[...]
    h = jnp.where(h >= 0, h, 0.01 * h)
    s = jnp.sum(h * w2_ref[...], axis=1) + b2_ref[0, 0]
    o_ref[0, 0, :] = s


def _score(x, W1, b1r, W2r, b2r):
    return pl.pallas_call(
        _score_body,
        grid=(NB,),
        in_specs=[
            pl.BlockSpec((RB, F), lambda i: (i, 0)),
            pl.BlockSpec((F, F), lambda i: (0, 0)),
            pl.BlockSpec((1, F), lambda i: (0, 0)),
            pl.BlockSpec((1, F), lambda i: (0, 0)),
            pl.BlockSpec((1, 1), lambda i: (0, 0)),
        ],
        out_specs=pl.BlockSpec((1, 1, RB), lambda i: (i, 0, 0)),
        out_shape=jax.ShapeDtypeStruct((NB, 1, RB), jnp.float32),
    )(x, W1, b1r, W2r, b2r)


def _sc_body(x_hbm, score_hbm, starts_hbm,
             maxp_hbm, sump_hbm, va_hbm, cnt_hbm, d_hbm,
             starts_v, xbuf, sbuf, stmax, stsum, stva, stcnt, std):
    wid = lax.axis_index("s") * 2 + lax.axis_index("c")
    seg_lo = wid * SEG_PER_W
    pltpu.sync_copy(starts_hbm.at[pl.ds(seg_lo, 48)], starts_v)

    neg_inf = jnp.full((16,), -jnp.inf, jnp.float32)
    zeros = jnp.zeros((16,), jnp.float32)

    def seg_body(s_rel, _):
        sv2 = starts_v[pl.ds(s_rel, 16)]
        r0 = sv2[0]
        r1 = sv2[1]
        r0a = (r0 // 8) * 8
        nch = jnp.where(r1 > r0, (r1 - r0a + C - 1) // C, 0)

        def chunk_body(j, carry):
            cb = r0a + j * C
            bb = jnp.minimum(cb, N - C)
            pltpu.sync_copy(x_hbm.at[pl.ds(bb * F, C * F)], xbuf)
            pltpu.sync_copy(score_hbm.at[pl.ds(bb, C)], sbuf.at[pl.ds(0, C)])
            lo = jnp.maximum(cb, r0) - bb
            hi = jnp.minimum(cb + C, r1) - bb

            def row_body(r, c):
                vmax, vsum, va, m, d = c
                sv = jnp.full((16,), sbuf[pl.ds(r, 16)][0], jnp.float32)
                mn = jnp.maximum(m, sv)
                eo = jnp.exp(m - mn)
                en = jnp.exp(sv - mn)
                d2 = d * eo + en
                xs = [xbuf[pl.ds(r * F + 16 * k, 16)] for k in range(8)]
                vmax2 = tuple(jnp.maximum(vmax[k], xs[k]) for k in range(8))
                vsum2 = tuple(vsum[k] + xs[k] for k in range(8))
                va2 = tuple(va[k] * eo + xs[k] * en for k in range(8))
                return (vmax2, vsum2, va2, mn, d2)

            return lax.fori_loop(lo, hi, row_body, carry)

        init = ((neg_inf,) * 8, (zeros,) * 8, (zeros,) * 8, neg_inf, zeros)
        vmax, vsum, va, m, d = lax.fori_loop(0, nch, chunk_body, init)

        cv = jnp.full((16,), (r1 - r0).astype(jnp.float32), jnp.float32)
        for k in range(8):
            stmax[pl.ds(s_rel * F + 16 * k, 16)] = vmax[k]
            stsum[pl.ds(s_rel * F + 16 * k, 16)] = vsum[k]
            stva[pl.ds(s_rel * F + 16 * k, 16)] = va[k]
        stcnt[pl.ds(s_rel * 16, 16)] = cv
        std[pl.ds(s_rel * 16, 16)] = d
        return 0

    lax.fori_loop(0, SEG_PER_W, seg_body, 0)

    pltpu.sync_copy(stmax, maxp_hbm.at[pl.ds(seg_lo * F, SEG_PER_W * F)])
    pltpu.sync_copy(stsum, sump_hbm.at[pl.ds(seg_lo * F, SEG_PER_W * F)])
    pltpu.sync_copy(stva, va_hbm.at[pl.ds(seg_lo * F, SEG_PER_W * F)])
    pltpu.sync_copy(stcnt, cnt_hbm.at[pl.ds(seg_lo * 16, SEG_PER_W * 16)])
    pltpu.sync_copy(std, d_hbm.at[pl.ds(seg_lo * 16, SEG_PER_W * 16)])


def _sc_reduce(x, score, starts):
    mesh = plsc.VectorSubcoreMesh(core_axis_name="c", subcore_axis_name="s")
    f32 = jnp.float32
    fn = functools.partial(
        pl.kernel,
        mesh=mesh,
        out_type=[
            jax.ShapeDtypeStruct((G * F,), f32),
            jax.ShapeDtypeStruct((G * F,), f32),
            jax.ShapeDtypeStruct((G * F,), f32),
            jax.ShapeDtypeStruct((G * 16,), f32),
            jax.ShapeDtypeStruct((G * 16,), f32),
        ],
        scratch_types=[
            pltpu.VMEM((48,), jnp.int32),
            pltpu.VMEM((C * F,), f32),
            pltpu.VMEM((C + 16,), f32),
            pltpu.VMEM((SEG_PER_W * F,), f32),
            pltpu.VMEM((SEG_PER_W * F,), f32),
            pltpu.VMEM((SEG_PER_W * F,), f32),
            pltpu.VMEM((SEG_PER_W * 16,), f32),
            pltpu.VMEM((SEG_PER_W * 16,), f32),
        ],
    )(_sc_body)
    maxp, sump, va, cnt, d = fn(x.reshape(N * F), score, starts)
    return (maxp.reshape(G, F), sump.reshape(G, F), va.reshape(G, F),
            cnt.reshape(G, 16), d.reshape(G, 16))


def _comb_body(maxp_ref, sump_ref, va_ref, cnt_ref, d_ref, wfc_ref, bfc_ref,
               o_ref):
    cnt = cnt_ref[...][:, 0:1]
    den = d_ref[...][:, 0:1]
    maxp = jnp.where(cnt > 0, maxp_ref[...], 0.0)
    sump = sump_ref[...]
    meanp = sump / jnp.maximum(cnt, 1.0)
    attn = va_ref[...] / (den + 1e-16)
    w = wfc_ref[...]
    out = jnp.dot(maxp, w[0:F], preferred_element_type=jnp.float32)
    out += jnp.dot(meanp, w[F:2 * F], preferred_element_type=jnp.float32)
    out += jnp.dot(sump, w[2 * F:3 * F], preferred_element_type=jnp.float32)
    out += jnp.dot(attn, w[3 * F:4 * F], preferred_element_type=jnp.float32)
    o_ref[...] = out + bfc_ref[...]


def _combine(maxp, sump, va, cnt, d, Wfc, bfcr):
    return pl.pallas_call(
        _comb_body,
        out_shape=jax.ShapeDtypeStruct((G, F), jnp.float32),
    )(maxp, sump, va, cnt, d, Wfc, bfcr)


def kernel(x, pos, batch, W1, b1, W2, b2, Wfc, bfc):
    del pos
    batch = batch.astype(jnp.int32)
    score2d = _score(x, W1, b1.reshape(1, F), W2.reshape(1, F),
                     b2.reshape(1, 1))
    score = score2d.reshape(N)
    starts = jnp.searchsorted(batch, jnp.arange(G + 1, dtype=jnp.int32))
    starts = jnp.concatenate(
        [starts.astype(jnp.int32),
         jnp.full((15,), N, jnp.int32)])
    maxp, sump, va, cnt, d = _sc_reduce(x, score, starts)
    out = _combine(maxp, sump, va, cnt, d, Wfc, bfc.reshape(1, F))
    return out[:1000]
